# baseline (speedup 1.0000x reference)
import dataclasses
import functools

import jax
import jax.numpy as jnp
from jax import lax
from jax.experimental import pallas as pl
from jax.experimental.pallas import tpu as pltpu
from jax.experimental.pallas import tpu_sc as plsc

N = 10000
D = 128
NPAD = 10240
NC, NS, L = 2, 16, 16
NW = NC * NS
C = 96
NQ = 6
Q = C // NQ
NV = 10112
ROWS_PER_TILE = NPAD // NS


def _cdiv(a, b):
    return (a + b - 1) // b


_GRID = 10
_RB = NPAD // _GRID


def _proj_body(x_ref, w_ref, av_s_ref, av_d_ref, h_ref, as_ref, ad_ref):
    h = jnp.dot(x_ref[...], w_ref[...], preferred_element_type=jnp.float32)
    h_ref[...] = h
    as_ref[...] = jnp.sum(h * av_s_ref[...][None, :], axis=1)
    ad_ref[...] = jnp.sum(h * av_d_ref[...][None, :], axis=1)


def _project(x, w, av_s, av_d):
    return pl.pallas_call(
        _proj_body,
        grid=(_GRID,),
        in_specs=[
            pl.BlockSpec((_RB, D), lambda i: (i, 0)),
            pl.BlockSpec((D, D), lambda i: (0, 0)),
            pl.BlockSpec((D,), lambda i: (0,)),
            pl.BlockSpec((D,), lambda i: (0,)),
        ],
        out_specs=[
            pl.BlockSpec((_RB, D), lambda i: (i, 0)),
            pl.BlockSpec((_RB,), lambda i: (i,)),
            pl.BlockSpec((_RB,), lambda i: (i,)),
        ],
        out_shape=[
            jax.ShapeDtypeStruct((NPAD, D), jnp.float32),
            jax.ShapeDtypeStruct((NPAD,), jnp.float32),
            jax.ShapeDtypeStruct((NPAD,), jnp.float32),
        ],
    )(x, w, av_s, av_d)


def _combine_body(accp_ref, sp_ref, b_ref, w_ref, av_s_ref, av_d_ref,
                  h_ref, as_ref, ad_ref):
    s = jnp.sum(sp_ref[...], axis=0)
    o = (accp_ref[0] + accp_ref[1]) / (s[:, None] + 1e-16) + b_ref[...][None, :]
    e = jnp.where(o > 0, o, jnp.exp(o) - 1.0)
    h = jnp.dot(e, w_ref[...], preferred_element_type=jnp.float32)
    h_ref[...] = h
    as_ref[...] = jnp.sum(h * av_s_ref[...][None, :], axis=1)
    ad_ref[...] = jnp.sum(h * av_d_ref[...][None, :], axis=1)


def _combine_project(accp, sp, b, w, av_s, av_d):
    return pl.pallas_call(
        _combine_body,
        grid=(_GRID,),
        in_specs=[
            pl.BlockSpec((NC, _RB, D), lambda i: (0, i, 0)),
            pl.BlockSpec((NW, _RB), lambda i: (0, i)),
            pl.BlockSpec((D,), lambda i: (0,)),
            pl.BlockSpec((D, D), lambda i: (0, 0)),
            pl.BlockSpec((D,), lambda i: (0,)),
            pl.BlockSpec((D,), lambda i: (0,)),
        ],
        out_specs=[
            pl.BlockSpec((_RB, D), lambda i: (i, 0)),
            pl.BlockSpec((_RB,), lambda i: (i,)),
            pl.BlockSpec((_RB,), lambda i: (i,)),
        ],
        out_shape=[
            jax.ShapeDtypeStruct((NPAD, D), jnp.float32),
            jax.ShapeDtypeStruct((NPAD,), jnp.float32),
            jax.ShapeDtypeStruct((NPAD,), jnp.float32),
        ],
    )(accp, sp, b, w, av_s, av_d)


def _final_body(accp_ref, sp_ref, b_ref, out_ref):
    s = jnp.sum(sp_ref[...], axis=0)
    out_ref[...] = ((accp_ref[0] + accp_ref[1]) / (s[:, None] + 1e-16)
                    + b_ref[...][None, :])


def _final(accp, sp, b):
    return pl.pallas_call(
        _final_body,
        grid=(_GRID,),
        in_specs=[
            pl.BlockSpec((NC, _RB, D), lambda i: (0, i, 0)),
            pl.BlockSpec((NW, _RB), lambda i: (0, i)),
            pl.BlockSpec((D,), lambda i: (0,)),
        ],
        out_specs=pl.BlockSpec((_RB, D), lambda i: (i, 0)),
        out_shape=jax.ShapeDtypeStruct((NPAD, D), jnp.float32),
    )(accp, sp, b)


def _make_edge_phase(e_real, gpt):
    ept = gpt * C
    mesh = plsc.VectorSubcoreMesh(core_axis_name="c", subcore_axis_name="s")
    cp = pltpu.CompilerParams()
    if "needs_layout_passes" in pltpu.CompilerParams.__dataclass_fields__:
        cp = dataclasses.replace(cp, needs_layout_passes=False)

    @functools.partial(
        pl.kernel,
        compiler_params=cp,
        out_type=[
            jax.ShapeDtypeStruct((NC, NPAD, D), jnp.float32),
            jax.ShapeDtypeStruct((NW, NPAD), jnp.float32),
        ],
        mesh=mesh,
        scratch_types=[
            pltpu.VMEM((NV,), jnp.float32),
            pltpu.VMEM((NV,), jnp.float32),
            pltpu.VMEM((NV,), jnp.float32),
            pltpu.VMEM((2 * NQ, Q), jnp.int32),
            pltpu.VMEM((2 * NQ, Q), jnp.int32),
            pltpu.VMEM((2 * NQ, Q), jnp.int32),
            pltpu.VMEM((C,), jnp.float32),
            pltpu.VMEM((Q, D), jnp.float32),
            pltpu.VMEM((Q, D), jnp.float32),
            pltpu.VMEM((Q, D), jnp.float32),
            pltpu.VMEM((Q, D), jnp.float32),
            pltpu.VMEM((Q, D), jnp.float32),
            pltpu.VMEM((Q, D), jnp.float32),
            pltpu.VMEM_SHARED((NPAD, D), jnp.float32),
            pltpu.SemaphoreType.DMA,
            pltpu.SemaphoreType.DMA,
            pltpu.SemaphoreType.DMA,
            pltpu.SemaphoreType.DMA,
            pltpu.SemaphoreType.DMA,
            pltpu.SemaphoreType.DMA,
            pltpu.SemaphoreType.DMA,
            pltpu.SemaphoreType.DMA,
            pltpu.SemaphoreType.DMA,
            pltpu.SemaphoreType.DMA,
            pltpu.SemaphoreType.DMA,
            pltpu.SemaphoreType.DMA,
            pltpu.SemaphoreType.DMA,
            pltpu.SemaphoreType.DMA,
            pltpu.SemaphoreType.DMA,
        ],
    )
    def edge_phase(h_hbm, asrc_hbm, adst_hbm, idx_hbm,
                   accp_hbm, sp_hbm,
                   as_v, ad_v, s_v, idx_0, idx_1, idx_2, w_buf,
                   rq_0, rq_1, rq_2, rq_3, rq_4, rq_5, acc,
                   isem_0, isem_1, isem_2,
                   gsem_0, gsem_1, gsem_2, gsem_3, gsem_4, gsem_5,
                   ssem_0, ssem_1, ssem_2, ssem_3, ssem_4, ssem_5):
        cid = lax.axis_index("c")
        sid = lax.axis_index("s")
        wid = cid * NS + sid
        rqs = (rq_0, rq_1, rq_2, rq_3, rq_4, rq_5)
        gsems = (gsem_0, gsem_1, gsem_2, gsem_3, gsem_4, gsem_5)
        ssems = (ssem_0, ssem_1, ssem_2, ssem_3, ssem_4, ssem_5)

        zeros16 = jnp.zeros((L,), jnp.float32)

        @pl.loop(0, Q)
        def _(j):
            for c in range(D // L):
                rq_0[j, pl.ds(c * L, L)] = zeros16

        @pl.loop(0, ROWS_PER_TILE // Q)
        def _(b):
            pltpu.sync_copy(rq_0,
                            acc.at[pl.ds(sid * ROWS_PER_TILE + b * Q, Q)])

        @pl.loop(0, NV // L)
        def _(j):
            s_v[pl.ds(j * L, L)] = zeros16

        pltpu.sync_copy(asrc_hbm.at[pl.ds(0, NV)], as_v)
        pltpu.sync_copy(adst_hbm.at[pl.ds(0, NV)], ad_v)
        pltpu.async_copy(idx_hbm.at[wid].at[0], idx_0, isem_0)
        pltpu.async_copy(idx_hbm.at[wid].at[1], idx_1, isem_1)

        plsc.subcore_barrier()

        lanes = lax.iota(jnp.int32, L)

        def scalar_phase(g, ibuf):
            base = wid * ept + g * C
            for k in range(NQ):
                sv = ibuf[k, pl.ds(0, L)]
                dv = ibuf[NQ + k, pl.ds(0, L)]
                av = plsc.load_gather(as_v, [sv])
                bv = plsc.load_gather(ad_v, [dv])
                e = av + bv
                e = jnp.where(e >= 0, e, e * jnp.float32(0.2))
                w = jnp.exp(e)
                valid = (base + k * L + lanes) < e_real
                w = jnp.where(valid, w, jnp.float32(0.0))
                w_buf[pl.ds(k * L, L)] = w
                plsc.addupdate_scatter(s_v, [dv], w)

        def multiply(i):
            @pl.loop(0, Q, unroll=4)
            def _(j):
                wj = plsc.load_gather(w_buf, [jnp.broadcast_to(j + i * Q,
                                                               (L,))])
                for c in range(D // L):
                    sl = pl.ds(c * L, L)
                    rqs[i % 6][j, sl] = rqs[i % 6][j, sl] * wj

        def wait_scatter(b, ibuf):
            pltpu.make_async_copy(rqs[b], acc.at[ibuf.at[NQ + b % NQ]],
                                  ssems[b]).wait()

        def issue_gather(b, ibuf, row):
            pltpu.async_copy(h_hbm.at[ibuf.at[row]], rqs[b], gsems[b])

        def finish_sixth(i, ibuf):
            pltpu.make_async_copy(h_hbm.at[ibuf.at[i]], rqs[i],
                                  gsems[i]).wait()
            multiply(i)
            pltpu.async_copy(rqs[i], acc.at[ibuf.at[NQ + i]], ssems[i],
                             add=True)

        def process_chunk(g, ibuf, nbuf, nsem, pbuf, psem):
            scalar_phase(g, ibuf)

            @pl.when(g > 0)
            def _():
                wait_scatter(4, ibuf)
            issue_gather(4, ibuf, 4)
            finish_sixth(0, ibuf)

            @pl.when(g > 0)
            def _():
                wait_scatter(5, ibuf)
            issue_gather(5, ibuf, 5)
            finish_sixth(1, ibuf)

            @pl.when(g + 2 < gpt)
            def _():
                pltpu.async_copy(idx_hbm.at[wid].at[g + 2], pbuf, psem)

            @pl.when(g + 1 < gpt)
            def _():
                pltpu.make_async_copy(idx_hbm.at[wid].at[g + 1], nbuf,
                                      nsem).wait()
                wait_scatter(0, ibuf)
                issue_gather(0, nbuf, 0)
            finish_sixth(2, ibuf)

            for i in range(3, 6):
                @pl.when(g + 1 < gpt)
                def _():
                    wait_scatter(i - 3 + 1, ibuf)
                    issue_gather(i - 3 + 1, nbuf, i - 3 + 1)
                finish_sixth(i, ibuf)

        pltpu.make_async_copy(idx_hbm.at[wid].at[0], idx_0, isem_0).wait()
        for b in range(4):
            issue_gather(b, idx_0, b)

        @pl.loop(0, gpt, step=3)
        def _(g):
            process_chunk(g, idx_0, idx_1, isem_1, idx_2, isem_2)
            process_chunk(g + 1, idx_1, idx_2, isem_2, idx_0, isem_0)
            process_chunk(g + 2, idx_2, idx_0, isem_0, idx_1, isem_1)

        for b in range(6):
            wait_scatter(b, idx_2)

        plsc.subcore_barrier()

        pltpu.sync_copy(s_v, sp_hbm.at[wid].at[pl.ds(0, NV)])
        pltpu.sync_copy(acc.at[pl.ds(sid * ROWS_PER_TILE, ROWS_PER_TILE)],
                        accp_hbm.at[cid].at[pl.ds(sid * ROWS_PER_TILE,
                                                  ROWS_PER_TILE)])

    return edge_phase


def kernel(x, edge_index, W1, a_src1, a_dst1, b1, W2, a_src2, a_dst2, b2):
    n = x.shape[0]
    e = edge_index.shape[1]
    e_real = e + n
    gpt = _cdiv(e_real, NW * C)
    gpt = _cdiv(gpt, 3) * 3
    epad = gpt * C * NW

    x_pad = jnp.pad(x, ((0, NPAD - n), (0, 0)))
    loop_idx = jnp.arange(n, dtype=edge_index.dtype)
    pad_idx = jnp.zeros((epad - e_real,), edge_index.dtype)
    src_t = jnp.concatenate([edge_index[0], loop_idx, pad_idx]).reshape(
        NW, gpt, NQ, Q)
    dst_t = jnp.concatenate([edge_index[1], loop_idx, pad_idx]).reshape(
        NW, gpt, NQ, Q)
    idx_t = jnp.concatenate([src_t, dst_t], axis=2)

    edge_phase = _make_edge_phase(e_real, gpt)

    h1, as1, ad1 = _project(x_pad, W1, a_src1.reshape(-1), a_dst1.reshape(-1))
    accp1, sp1 = edge_phase(h1, as1, ad1, idx_t)
    h2, as2, ad2 = _combine_project(accp1, sp1, b1, W2,
                                    a_src2.reshape(-1), a_dst2.reshape(-1))
    accp2, sp2 = edge_phase(h2, as2, ad2, idx_t)
    out = _final(accp2, sp2, b2)
    return out[:n]


# speedup vs baseline: 1.1438x; 1.1438x over previous
import dataclasses
import functools

import jax
import jax.numpy as jnp
from jax import lax
from jax.experimental import pallas as pl
from jax.experimental.pallas import tpu as pltpu
from jax.experimental.pallas import tpu_sc as plsc

N = 10000
D = 128
NPAD = 10240
NC, NS, L = 2, 16, 16
NW = NC * NS
C = 96
NQ = 3
Q = C // NQ
NV = 10112
ROWS_PER_TILE = NPAD // NS


def _cdiv(a, b):
    return (a + b - 1) // b


_GRID = 10
_RB = NPAD // _GRID


def _proj_body(x_ref, w_ref, av_s_ref, av_d_ref, h_ref, as_ref, ad_ref):
    h = jnp.dot(x_ref[...], w_ref[...], preferred_element_type=jnp.float32)
    h_ref[...] = h
    as_ref[...] = jnp.sum(h * av_s_ref[...][None, :], axis=1)
    ad_ref[...] = jnp.sum(h * av_d_ref[...][None, :], axis=1)


def _project(x, w, av_s, av_d):
    return pl.pallas_call(
        _proj_body,
        grid=(_GRID,),
        in_specs=[
            pl.BlockSpec((_RB, D), lambda i: (i, 0)),
            pl.BlockSpec((D, D), lambda i: (0, 0)),
            pl.BlockSpec((D,), lambda i: (0,)),
            pl.BlockSpec((D,), lambda i: (0,)),
        ],
        out_specs=[
            pl.BlockSpec((_RB, D), lambda i: (i, 0)),
            pl.BlockSpec((_RB,), lambda i: (i,)),
            pl.BlockSpec((_RB,), lambda i: (i,)),
        ],
        out_shape=[
            jax.ShapeDtypeStruct((NPAD, D), jnp.float32),
            jax.ShapeDtypeStruct((NPAD,), jnp.float32),
            jax.ShapeDtypeStruct((NPAD,), jnp.float32),
        ],
    )(x, w, av_s, av_d)


def _combine_body(accp_ref, sp_ref, b_ref, w_ref, av_s_ref, av_d_ref,
                  h_ref, as_ref, ad_ref):
    s = jnp.sum(sp_ref[...], axis=0)
    o = (accp_ref[0] + accp_ref[1]) / (s[:, None] + 1e-16) + b_ref[...][None, :]
    e = jnp.where(o > 0, o, jnp.exp(o) - 1.0)
    h = jnp.dot(e, w_ref[...], preferred_element_type=jnp.float32)
    h_ref[...] = h
    as_ref[...] = jnp.sum(h * av_s_ref[...][None, :], axis=1)
    ad_ref[...] = jnp.sum(h * av_d_ref[...][None, :], axis=1)


def _combine_project(accp, sp, b, w, av_s, av_d):
    return pl.pallas_call(
        _combine_body,
        grid=(_GRID,),
        in_specs=[
            pl.BlockSpec((NC, _RB, D), lambda i: (0, i, 0)),
            pl.BlockSpec((NW, _RB), lambda i: (0, i)),
            pl.BlockSpec((D,), lambda i: (0,)),
            pl.BlockSpec((D, D), lambda i: (0, 0)),
            pl.BlockSpec((D,), lambda i: (0,)),
            pl.BlockSpec((D,), lambda i: (0,)),
        ],
        out_specs=[
            pl.BlockSpec((_RB, D), lambda i: (i, 0)),
            pl.BlockSpec((_RB,), lambda i: (i,)),
            pl.BlockSpec((_RB,), lambda i: (i,)),
        ],
        out_shape=[
            jax.ShapeDtypeStruct((NPAD, D), jnp.float32),
            jax.ShapeDtypeStruct((NPAD,), jnp.float32),
            jax.ShapeDtypeStruct((NPAD,), jnp.float32),
        ],
    )(accp, sp, b, w, av_s, av_d)


def _final_body(accp_ref, sp_ref, b_ref, out_ref):
    s = jnp.sum(sp_ref[...], axis=0)
    out_ref[...] = ((accp_ref[0] + accp_ref[1]) / (s[:, None] + 1e-16)
                    + b_ref[...][None, :])


def _final(accp, sp, b):
    return pl.pallas_call(
        _final_body,
        grid=(_GRID,),
        in_specs=[
            pl.BlockSpec((NC, _RB, D), lambda i: (0, i, 0)),
            pl.BlockSpec((NW, _RB), lambda i: (0, i)),
            pl.BlockSpec((D,), lambda i: (0,)),
        ],
        out_specs=pl.BlockSpec((_RB, D), lambda i: (i, 0)),
        out_shape=jax.ShapeDtypeStruct((NPAD, D), jnp.float32),
    )(accp, sp, b)


def _make_edge_phase(e_real, gpt):
    ept = gpt * C
    mesh = plsc.VectorSubcoreMesh(core_axis_name="c", subcore_axis_name="s")
    cp = pltpu.CompilerParams()
    if "needs_layout_passes" in pltpu.CompilerParams.__dataclass_fields__:
        cp = dataclasses.replace(cp, needs_layout_passes=False)

    @functools.partial(
        pl.kernel,
        compiler_params=cp,
        out_type=[
            jax.ShapeDtypeStruct((NC, NPAD, D), jnp.float32),
            jax.ShapeDtypeStruct((NW, NPAD), jnp.float32),
        ],
        mesh=mesh,
        scratch_types=[
            pltpu.VMEM((NV,), jnp.float32),
            pltpu.VMEM((NV,), jnp.float32),
            pltpu.VMEM((NV,), jnp.float32),
            pltpu.VMEM((2 * NQ, Q), jnp.int32),
            pltpu.VMEM((2 * NQ, Q), jnp.int32),
            pltpu.VMEM((2 * NQ, Q), jnp.int32),
            pltpu.VMEM((C,), jnp.float32),
            pltpu.VMEM((Q, D), jnp.float32),
            pltpu.VMEM((Q, D), jnp.float32),
            pltpu.VMEM((Q, D), jnp.float32),
            pltpu.VMEM_SHARED((NPAD, D), jnp.float32),
            pltpu.SemaphoreType.DMA,
            pltpu.SemaphoreType.DMA,
            pltpu.SemaphoreType.DMA,
            pltpu.SemaphoreType.DMA,
            pltpu.SemaphoreType.DMA,
            pltpu.SemaphoreType.DMA,
            pltpu.SemaphoreType.DMA,
            pltpu.SemaphoreType.DMA,
            pltpu.SemaphoreType.DMA,
        ],
    )
    def edge_phase(h_hbm, asrc_hbm, adst_hbm, idx_hbm,
                   accp_hbm, sp_hbm,
                   as_v, ad_v, s_v, idx_0, idx_1, idx_2, w_buf,
                   rq_0, rq_1, rq_2, acc,
                   isem_0, isem_1, isem_2,
                   gsem_0, gsem_1, gsem_2,
                   ssem_0, ssem_1, ssem_2):
        cid = lax.axis_index("c")
        sid = lax.axis_index("s")
        wid = cid * NS + sid
        rqs = (rq_0, rq_1, rq_2)
        gsems = (gsem_0, gsem_1, gsem_2)
        ssems = (ssem_0, ssem_1, ssem_2)

        zeros16 = jnp.zeros((L,), jnp.float32)

        @pl.loop(0, Q)
        def _(j):
            for c in range(D // L):
                rq_0[j, pl.ds(c * L, L)] = zeros16

        @pl.loop(0, ROWS_PER_TILE // Q)
        def _(b):
            pltpu.sync_copy(rq_0,
                            acc.at[pl.ds(sid * ROWS_PER_TILE + b * Q, Q)])

        _rem = ROWS_PER_TILE % Q
        if _rem:
            pltpu.sync_copy(
                rq_0.at[pl.ds(0, _rem)],
                acc.at[pl.ds(sid * ROWS_PER_TILE + ROWS_PER_TILE - _rem,
                             _rem)])

        @pl.loop(0, NV // L)
        def _(j):
            s_v[pl.ds(j * L, L)] = zeros16

        pltpu.sync_copy(asrc_hbm.at[pl.ds(0, NV)], as_v)
        pltpu.sync_copy(adst_hbm.at[pl.ds(0, NV)], ad_v)
        pltpu.async_copy(idx_hbm.at[wid].at[0], idx_0, isem_0)
        pltpu.async_copy(idx_hbm.at[wid].at[1], idx_1, isem_1)

        plsc.subcore_barrier()

        lanes = lax.iota(jnp.int32, L)

        def scalar_phase(g, ibuf):
            base = wid * ept + g * C
            for k in range(C // L):
                row, off = (k * L) // Q, (k * L) % Q
                sv = ibuf[row, pl.ds(off, L)]
                dv = ibuf[NQ + row, pl.ds(off, L)]
                av = plsc.load_gather(as_v, [sv])
                bv = plsc.load_gather(ad_v, [dv])
                e = av + bv
                e = jnp.where(e >= 0, e, e * jnp.float32(0.2))
                w = jnp.exp(e)
                valid = (base + k * L + lanes) < e_real
                w = jnp.where(valid, w, jnp.float32(0.0))
                w_buf[pl.ds(k * L, L)] = w
                plsc.addupdate_scatter(s_v, [dv], w)

        def multiply(i):
            @plsc.parallel_loop(0, Q, unroll=4)
            def _(j):
                wj = plsc.load_gather(w_buf, [jnp.broadcast_to(j + i * Q,
                                                               (L,))])
                for c in range(D // L):
                    sl = pl.ds(c * L, L)
                    rqs[i][j, sl] = rqs[i][j, sl] * wj

        def wait_scatter(i, ibuf):
            pltpu.make_async_copy(rqs[i], acc.at[ibuf.at[NQ + i]],
                                  ssems[i]).wait()

        def issue_gather(i, ibuf):
            pltpu.async_copy(h_hbm.at[ibuf.at[i]], rqs[i], gsems[i])

        def finish_third(i, ibuf):
            pltpu.make_async_copy(h_hbm.at[ibuf.at[i]], rqs[i],
                                  gsems[i]).wait()
            multiply(i)
            pltpu.async_copy(rqs[i], acc.at[ibuf.at[NQ + i]], ssems[i],
                             add=True)

        def process_chunk(g, ibuf, nbuf, nsem, pbuf, psem):
            scalar_phase(g, ibuf)

            @pl.when(g > 0)
            def _():
                wait_scatter(2, ibuf)
            issue_gather(2, ibuf)

            @pl.when(g + 2 < gpt)
            def _():
                pltpu.async_copy(idx_hbm.at[wid].at[g + 2], pbuf, psem)

            finish_third(0, ibuf)
            finish_third(1, ibuf)

            @pl.when(g + 1 < gpt)
            def _():
                pltpu.make_async_copy(idx_hbm.at[wid].at[g + 1], nbuf,
                                      nsem).wait()
                wait_scatter(0, nbuf)
                issue_gather(0, nbuf)
                wait_scatter(1, nbuf)
                issue_gather(1, nbuf)

            finish_third(2, ibuf)

        pltpu.make_async_copy(idx_hbm.at[wid].at[0], idx_0, isem_0).wait()
        issue_gather(0, idx_0)
        issue_gather(1, idx_0)

        @pl.loop(0, gpt, step=3)
        def _(g):
            process_chunk(g, idx_0, idx_1, isem_1, idx_2, isem_2)
            process_chunk(g + 1, idx_1, idx_2, isem_2, idx_0, isem_0)
            process_chunk(g + 2, idx_2, idx_0, isem_0, idx_1, isem_1)

        for i in range(NQ):
            wait_scatter(i, idx_2)

        plsc.subcore_barrier()

        pltpu.sync_copy(s_v, sp_hbm.at[wid].at[pl.ds(0, NV)])
        pltpu.sync_copy(acc.at[pl.ds(sid * ROWS_PER_TILE, ROWS_PER_TILE)],
                        accp_hbm.at[cid].at[pl.ds(sid * ROWS_PER_TILE,
                                                  ROWS_PER_TILE)])

    return edge_phase


def kernel(x, edge_index, W1, a_src1, a_dst1, b1, W2, a_src2, a_dst2, b2):
    n = x.shape[0]
    e = edge_index.shape[1]
    e_real = e + n
    gpt = _cdiv(e_real, NW * C)
    gpt = _cdiv(gpt, 3) * 3
    epad = gpt * C * NW

    x_pad = jnp.pad(x, ((0, NPAD - n), (0, 0)))
    loop_idx = jnp.arange(n, dtype=edge_index.dtype)
    pad_idx = jnp.zeros((epad - e_real,), edge_index.dtype)
    src_t = jnp.concatenate([edge_index[0], loop_idx, pad_idx]).reshape(
        NW, gpt, NQ, Q)
    dst_t = jnp.concatenate([edge_index[1], loop_idx, pad_idx]).reshape(
        NW, gpt, NQ, Q)
    idx_t = jnp.concatenate([src_t, dst_t], axis=2)

    edge_phase = _make_edge_phase(e_real, gpt)

    h1, as1, ad1 = _project(x_pad, W1, a_src1.reshape(-1), a_dst1.reshape(-1))
    accp1, sp1 = edge_phase(h1, as1, ad1, idx_t)
    h2, as2, ad2 = _combine_project(accp1, sp1, b1, W2,
                                    a_src2.reshape(-1), a_dst2.reshape(-1))
    accp2, sp2 = edge_phase(h2, as2, ad2, idx_t)
    out = _final(accp2, sp2, b2)
    return out[:n]


# speedup vs baseline: 1.1884x; 1.0390x over previous
import dataclasses
import functools

import jax
import jax.numpy as jnp
from jax import lax
from jax.experimental import pallas as pl
from jax.experimental.pallas import tpu as pltpu
from jax.experimental.pallas import tpu_sc as plsc

N = 10000
D = 128
NPAD = 10240
NC, NS, L = 2, 16, 16
NW = NC * NS
C = 128
NQ = 4
Q = C // NQ
NV = 10112
ROWS_PER_TILE = NPAD // NS


def _cdiv(a, b):
    return (a + b - 1) // b


_GRID = 10
_RB = NPAD // _GRID


def _proj_body(x_ref, w_ref, av_s_ref, av_d_ref, h_ref, as_ref, ad_ref):
    h = jnp.dot(x_ref[...], w_ref[...], preferred_element_type=jnp.float32)
    h_ref[...] = h
    as_ref[...] = jnp.sum(h * av_s_ref[...][None, :], axis=1)
    ad_ref[...] = jnp.sum(h * av_d_ref[...][None, :], axis=1)


def _project(x, w, av_s, av_d):
    return pl.pallas_call(
        _proj_body,
        grid=(_GRID,),
        in_specs=[
            pl.BlockSpec((_RB, D), lambda i: (i, 0)),
            pl.BlockSpec((D, D), lambda i: (0, 0)),
            pl.BlockSpec((D,), lambda i: (0,)),
            pl.BlockSpec((D,), lambda i: (0,)),
        ],
        out_specs=[
            pl.BlockSpec((_RB, D), lambda i: (i, 0)),
            pl.BlockSpec((_RB,), lambda i: (i,)),
            pl.BlockSpec((_RB,), lambda i: (i,)),
        ],
        out_shape=[
            jax.ShapeDtypeStruct((NPAD, D), jnp.float32),
            jax.ShapeDtypeStruct((NPAD,), jnp.float32),
            jax.ShapeDtypeStruct((NPAD,), jnp.float32),
        ],
    )(x, w, av_s, av_d)


def _combine_body(accp_ref, sp_ref, b_ref, w_ref, av_s_ref, av_d_ref,
                  h_ref, as_ref, ad_ref):
    s = jnp.sum(sp_ref[...], axis=0)
    o = (accp_ref[0] + accp_ref[1]) / (s[:, None] + 1e-16) + b_ref[...][None, :]
    e = jnp.where(o > 0, o, jnp.exp(o) - 1.0)
    h = jnp.dot(e, w_ref[...], preferred_element_type=jnp.float32)
    h_ref[...] = h
    as_ref[...] = jnp.sum(h * av_s_ref[...][None, :], axis=1)
    ad_ref[...] = jnp.sum(h * av_d_ref[...][None, :], axis=1)


def _combine_project(accp, sp, b, w, av_s, av_d):
    return pl.pallas_call(
        _combine_body,
        grid=(_GRID,),
        in_specs=[
            pl.BlockSpec((NC, _RB, D), lambda i: (0, i, 0)),
            pl.BlockSpec((NW, _RB), lambda i: (0, i)),
            pl.BlockSpec((D,), lambda i: (0,)),
            pl.BlockSpec((D, D), lambda i: (0, 0)),
            pl.BlockSpec((D,), lambda i: (0,)),
            pl.BlockSpec((D,), lambda i: (0,)),
        ],
        out_specs=[
            pl.BlockSpec((_RB, D), lambda i: (i, 0)),
            pl.BlockSpec((_RB,), lambda i: (i,)),
            pl.BlockSpec((_RB,), lambda i: (i,)),
        ],
        out_shape=[
            jax.ShapeDtypeStruct((NPAD, D), jnp.float32),
            jax.ShapeDtypeStruct((NPAD,), jnp.float32),
            jax.ShapeDtypeStruct((NPAD,), jnp.float32),
        ],
    )(accp, sp, b, w, av_s, av_d)


def _final_body(accp_ref, sp_ref, b_ref, out_ref):
    s = jnp.sum(sp_ref[...], axis=0)
    out_ref[...] = ((accp_ref[0] + accp_ref[1]) / (s[:, None] + 1e-16)
                    + b_ref[...][None, :])


def _final(accp, sp, b):
    return pl.pallas_call(
        _final_body,
        grid=(_GRID,),
        in_specs=[
            pl.BlockSpec((NC, _RB, D), lambda i: (0, i, 0)),
            pl.BlockSpec((NW, _RB), lambda i: (0, i)),
            pl.BlockSpec((D,), lambda i: (0,)),
        ],
        out_specs=pl.BlockSpec((_RB, D), lambda i: (i, 0)),
        out_shape=jax.ShapeDtypeStruct((NPAD, D), jnp.float32),
    )(accp, sp, b)


def _make_edge_phase(e_real, gpt):
    ept = gpt * C
    mesh = plsc.VectorSubcoreMesh(core_axis_name="c", subcore_axis_name="s")
    cp = pltpu.CompilerParams()
    if "needs_layout_passes" in pltpu.CompilerParams.__dataclass_fields__:
        cp = dataclasses.replace(cp, needs_layout_passes=False)

    @functools.partial(
        pl.kernel,
        compiler_params=cp,
        out_type=[
            jax.ShapeDtypeStruct((NC, NPAD, D), jnp.float32),
            jax.ShapeDtypeStruct((NW, NPAD), jnp.float32),
        ],
        mesh=mesh,
        scratch_types=[
            pltpu.VMEM((NV,), jnp.float32),
            pltpu.VMEM((NV,), jnp.float32),
            pltpu.VMEM((NV,), jnp.float32),
            pltpu.VMEM((2 * NQ, Q), jnp.int32),
            pltpu.VMEM((2 * NQ, Q), jnp.int32),
            pltpu.VMEM((2 * NQ, Q), jnp.int32),
            pltpu.VMEM((C,), jnp.float32),
            pltpu.VMEM((Q, D), jnp.float32),
            pltpu.VMEM((Q, D), jnp.float32),
            pltpu.VMEM((Q, D), jnp.float32),
            pltpu.VMEM((Q, D), jnp.float32),
            pltpu.VMEM_SHARED((NV, D), jnp.float32),
            pltpu.SemaphoreType.DMA,
            pltpu.SemaphoreType.DMA,
            pltpu.SemaphoreType.DMA,
            pltpu.SemaphoreType.DMA,
            pltpu.SemaphoreType.DMA,
            pltpu.SemaphoreType.DMA,
            pltpu.SemaphoreType.DMA,
            pltpu.SemaphoreType.DMA,
            pltpu.SemaphoreType.DMA,
            pltpu.SemaphoreType.DMA,
            pltpu.SemaphoreType.DMA,
        ],
    )
    def edge_phase(h_hbm, asrc_hbm, adst_hbm, idx_hbm,
                   accp_hbm, sp_hbm,
                   as_v, ad_v, s_v, idx_0, idx_1, idx_2, w_buf,
                   rq_0, rq_1, rq_2, rq_3, acc,
                   isem_0, isem_1, isem_2,
                   gsem_0, gsem_1, gsem_2, gsem_3,
                   ssem_0, ssem_1, ssem_2, ssem_3):
        cid = lax.axis_index("c")
        sid = lax.axis_index("s")
        wid = cid * NS + sid
        rqs = (rq_0, rq_1, rq_2, rq_3)
        gsems = (gsem_0, gsem_1, gsem_2, gsem_3)
        ssems = (ssem_0, ssem_1, ssem_2, ssem_3)

        zeros16 = jnp.zeros((L,), jnp.float32)

        @pl.loop(0, Q)
        def _(j):
            for c in range(D // L):
                rq_0[j, pl.ds(c * L, L)] = zeros16

        acc_rpt = NV // NS

        @pl.loop(0, acc_rpt // Q)
        def _(b):
            pltpu.sync_copy(rq_0, acc.at[pl.ds(sid * acc_rpt + b * Q, Q)])

        _rem = acc_rpt % Q
        if _rem:
            pltpu.sync_copy(
                rq_0.at[pl.ds(0, _rem)],
                acc.at[pl.ds(sid * acc_rpt + acc_rpt - _rem, _rem)])

        @pl.loop(0, NV // L)
        def _(j):
            s_v[pl.ds(j * L, L)] = zeros16

        pltpu.sync_copy(asrc_hbm.at[pl.ds(0, NV)], as_v)
        pltpu.sync_copy(adst_hbm.at[pl.ds(0, NV)], ad_v)
        pltpu.async_copy(idx_hbm.at[wid].at[0], idx_0, isem_0)
        pltpu.async_copy(idx_hbm.at[wid].at[1], idx_1, isem_1)

        plsc.subcore_barrier()

        lanes = lax.iota(jnp.int32, L)

        def scalar_phase(g, ibuf):
            base = wid * ept + g * C
            for k in range(C // L):
                row, off = (k * L) // Q, (k * L) % Q
                sv = ibuf[row, pl.ds(off, L)]
                dv = ibuf[NQ + row, pl.ds(off, L)]
                av = plsc.load_gather(as_v, [sv])
                bv = plsc.load_gather(ad_v, [dv])
                e = av + bv
                e = jnp.where(e >= 0, e, e * jnp.float32(0.2))
                w = jnp.exp(e)
                valid = (base + k * L + lanes) < e_real
                w = jnp.where(valid, w, jnp.float32(0.0))
                w_buf[pl.ds(k * L, L)] = w
                plsc.addupdate_scatter(s_v, [dv], w)

        def multiply(i):
            @plsc.parallel_loop(0, Q, unroll=4)
            def _(j):
                wj = plsc.load_gather(w_buf, [jnp.broadcast_to(j + i * Q,
                                                               (L,))])
                for c in range(D // L):
                    sl = pl.ds(c * L, L)
                    rqs[i][j, sl] = rqs[i][j, sl] * wj

        def wait_scatter(i, ibuf):
            pltpu.make_async_copy(rqs[i], acc.at[ibuf.at[NQ + i]],
                                  ssems[i]).wait()

        def issue_gather(i, ibuf):
            pltpu.async_copy(h_hbm.at[ibuf.at[i]], rqs[i], gsems[i])

        def finish_quarter(i, ibuf):
            pltpu.make_async_copy(h_hbm.at[ibuf.at[i]], rqs[i],
                                  gsems[i]).wait()
            multiply(i)
            pltpu.async_copy(rqs[i], acc.at[ibuf.at[NQ + i]], ssems[i],
                             add=True)

        def process_chunk(g, ibuf, nbuf, nsem, pbuf, psem):
            scalar_phase(g, ibuf)

            @pl.when(g > 0)
            def _():
                wait_scatter(3, ibuf)
            issue_gather(3, ibuf)

            @pl.when(g + 2 < gpt)
            def _():
                pltpu.async_copy(idx_hbm.at[wid].at[g + 2], pbuf, psem)

            finish_quarter(0, ibuf)

            @pl.when(g + 1 < gpt)
            def _():
                pltpu.make_async_copy(idx_hbm.at[wid].at[g + 1], nbuf,
                                      nsem).wait()
                wait_scatter(0, nbuf)
                issue_gather(0, nbuf)

            finish_quarter(1, ibuf)

            @pl.when(g + 1 < gpt)
            def _():
                wait_scatter(1, nbuf)
                issue_gather(1, nbuf)

            finish_quarter(2, ibuf)

            @pl.when(g + 1 < gpt)
            def _():
                wait_scatter(2, nbuf)
                issue_gather(2, nbuf)

            finish_quarter(3, ibuf)

        pltpu.make_async_copy(idx_hbm.at[wid].at[0], idx_0, isem_0).wait()
        issue_gather(0, idx_0)
        issue_gather(1, idx_0)
        issue_gather(2, idx_0)

        @pl.loop(0, gpt, step=3)
        def _(g):
            process_chunk(g, idx_0, idx_1, isem_1, idx_2, isem_2)
            process_chunk(g + 1, idx_1, idx_2, isem_2, idx_0, isem_0)
            process_chunk(g + 2, idx_2, idx_0, isem_0, idx_1, isem_1)

        for i in range(NQ):
            wait_scatter(i, idx_2)

        plsc.subcore_barrier()

        pltpu.sync_copy(s_v, sp_hbm.at[wid].at[pl.ds(0, NV)])
        pltpu.sync_copy(acc.at[pl.ds(sid * acc_rpt, acc_rpt)],
                        accp_hbm.at[cid].at[pl.ds(sid * acc_rpt, acc_rpt)])

    return edge_phase


def kernel(x, edge_index, W1, a_src1, a_dst1, b1, W2, a_src2, a_dst2, b2):
    n = x.shape[0]
    e = edge_index.shape[1]
    e_real = e + n
    gpt = _cdiv(e_real, NW * C)
    gpt = _cdiv(gpt, 3) * 3
    epad = gpt * C * NW

    x_pad = jnp.pad(x, ((0, NPAD - n), (0, 0)))
    loop_idx = jnp.arange(n, dtype=edge_index.dtype)
    pad_idx = jnp.zeros((epad - e_real,), edge_index.dtype)
    src_t = jnp.concatenate([edge_index[0], loop_idx, pad_idx]).reshape(
        NW, gpt, NQ, Q)
    dst_t = jnp.concatenate([edge_index[1], loop_idx, pad_idx]).reshape(
        NW, gpt, NQ, Q)
    idx_t = jnp.concatenate([src_t, dst_t], axis=2)

    edge_phase = _make_edge_phase(e_real, gpt)

    h1, as1, ad1 = _project(x_pad, W1, a_src1.reshape(-1), a_dst1.reshape(-1))
    accp1, sp1 = edge_phase(h1, as1, ad1, idx_t)
    h2, as2, ad2 = _combine_project(accp1, sp1, b1, W2,
                                    a_src2.reshape(-1), a_dst2.reshape(-1))
    accp2, sp2 = edge_phase(h2, as2, ad2, idx_t)
    out = _final(accp2, sp2, b2)
    return out[:n]


# speedup vs baseline: 1.2183x; 1.0251x over previous
import dataclasses
import functools

import jax
import jax.numpy as jnp
from jax import lax
from jax.experimental import pallas as pl
from jax.experimental.pallas import tpu as pltpu
from jax.experimental.pallas import tpu_sc as plsc

N = 10000
D = 128
NPAD = 10240
NC, NS, L = 2, 16, 16
NW = NC * NS
C = 128
NQ = 4
Q = C // NQ
NV = 10112
ROWS_PER_TILE = NPAD // NS


def _cdiv(a, b):
    return (a + b - 1) // b


_GRID = 10
_RB = NPAD // _GRID


def _proj_body(x_ref, w_ref, av_s_ref, av_d_ref, h_ref, as_ref, ad_ref):
    h = jnp.dot(x_ref[...], w_ref[...], preferred_element_type=jnp.float32)
    h_ref[...] = h
    as_ref[...] = jnp.sum(h * av_s_ref[...][None, :], axis=1)
    ad_ref[...] = jnp.sum(h * av_d_ref[...][None, :], axis=1)


def _project(x, w, av_s, av_d):
    return pl.pallas_call(
        _proj_body,
        grid=(_GRID,),
        in_specs=[
            pl.BlockSpec((_RB, D), lambda i: (i, 0)),
            pl.BlockSpec((D, D), lambda i: (0, 0)),
            pl.BlockSpec((D,), lambda i: (0,)),
            pl.BlockSpec((D,), lambda i: (0,)),
        ],
        out_specs=[
            pl.BlockSpec((_RB, D), lambda i: (i, 0)),
            pl.BlockSpec((_RB,), lambda i: (i,)),
            pl.BlockSpec((_RB,), lambda i: (i,)),
        ],
        out_shape=[
            jax.ShapeDtypeStruct((NPAD, D), jnp.float32),
            jax.ShapeDtypeStruct((NPAD,), jnp.float32),
            jax.ShapeDtypeStruct((NPAD,), jnp.float32),
        ],
    )(x, w, av_s, av_d)


def _combine_body(accp_ref, sp_ref, b_ref, w_ref, av_s_ref, av_d_ref,
                  h_ref, as_ref, ad_ref):
    s = jnp.sum(sp_ref[...], axis=0)
    o = (accp_ref[0] + accp_ref[1]) / (s[:, None] + 1e-16) + b_ref[...][None, :]
    e = jnp.where(o > 0, o, jnp.exp(o) - 1.0)
    h = jnp.dot(e, w_ref[...], preferred_element_type=jnp.float32)
    h_ref[...] = h
    as_ref[...] = jnp.sum(h * av_s_ref[...][None, :], axis=1)
    ad_ref[...] = jnp.sum(h * av_d_ref[...][None, :], axis=1)


def _combine_project(accp, sp, b, w, av_s, av_d):
    return pl.pallas_call(
        _combine_body,
        grid=(_GRID,),
        in_specs=[
            pl.BlockSpec((NC, _RB, D), lambda i: (0, i, 0)),
            pl.BlockSpec((NW, _RB), lambda i: (0, i)),
            pl.BlockSpec((D,), lambda i: (0,)),
            pl.BlockSpec((D, D), lambda i: (0, 0)),
            pl.BlockSpec((D,), lambda i: (0,)),
            pl.BlockSpec((D,), lambda i: (0,)),
        ],
        out_specs=[
            pl.BlockSpec((_RB, D), lambda i: (i, 0)),
            pl.BlockSpec((_RB,), lambda i: (i,)),
            pl.BlockSpec((_RB,), lambda i: (i,)),
        ],
        out_shape=[
            jax.ShapeDtypeStruct((NPAD, D), jnp.float32),
            jax.ShapeDtypeStruct((NPAD,), jnp.float32),
            jax.ShapeDtypeStruct((NPAD,), jnp.float32),
        ],
    )(accp, sp, b, w, av_s, av_d)


def _final_body(accp_ref, sp_ref, b_ref, out_ref):
    s = jnp.sum(sp_ref[...], axis=0)
    out_ref[...] = ((accp_ref[0] + accp_ref[1]) / (s[:, None] + 1e-16)
                    + b_ref[...][None, :])


def _final(accp, sp, b):
    return pl.pallas_call(
        _final_body,
        grid=(_GRID,),
        in_specs=[
            pl.BlockSpec((NC, _RB, D), lambda i: (0, i, 0)),
            pl.BlockSpec((NW, _RB), lambda i: (0, i)),
            pl.BlockSpec((D,), lambda i: (0,)),
        ],
        out_specs=pl.BlockSpec((_RB, D), lambda i: (i, 0)),
        out_shape=jax.ShapeDtypeStruct((NPAD, D), jnp.float32),
    )(accp, sp, b)


def _make_edge_phase(e_real, gpt0, gpt1):
    mesh = plsc.VectorSubcoreMesh(core_axis_name="c", subcore_axis_name="s")
    cp = pltpu.CompilerParams()
    if "needs_layout_passes" in pltpu.CompilerParams.__dataclass_fields__:
        cp = dataclasses.replace(cp, needs_layout_passes=False)

    @functools.partial(
        pl.kernel,
        compiler_params=cp,
        out_type=[
            jax.ShapeDtypeStruct((NC, NPAD, D), jnp.float32),
            jax.ShapeDtypeStruct((NW, NPAD), jnp.float32),
        ],
        mesh=mesh,
        scratch_types=[
            pltpu.VMEM((NV,), jnp.float32),
            pltpu.VMEM((NV,), jnp.float32),
            pltpu.VMEM((NV,), jnp.float32),
            pltpu.VMEM((2 * NQ, Q), jnp.int32),
            pltpu.VMEM((2 * NQ, Q), jnp.int32),
            pltpu.VMEM((2 * NQ, Q), jnp.int32),
            pltpu.VMEM((C,), jnp.float32),
            pltpu.VMEM((Q, D), jnp.float32),
            pltpu.VMEM((Q, D), jnp.float32),
            pltpu.VMEM((Q, D), jnp.float32),
            pltpu.VMEM((Q, D), jnp.float32),
            pltpu.VMEM_SHARED((NV, D), jnp.float32),
            pltpu.SemaphoreType.DMA,
            pltpu.SemaphoreType.DMA,
            pltpu.SemaphoreType.DMA,
            pltpu.SemaphoreType.DMA,
            pltpu.SemaphoreType.DMA,
            pltpu.SemaphoreType.DMA,
            pltpu.SemaphoreType.DMA,
            pltpu.SemaphoreType.DMA,
            pltpu.SemaphoreType.DMA,
            pltpu.SemaphoreType.DMA,
            pltpu.SemaphoreType.DMA,
        ],
    )
    def edge_phase(h_hbm, asrc_hbm, adst_hbm, idx_hbm,
                   accp_hbm, sp_hbm,
                   as_v, ad_v, s_v, idx_0, idx_1, idx_2, w_buf,
                   rq_0, rq_1, rq_2, rq_3, acc,
                   isem_0, isem_1, isem_2,
                   gsem_0, gsem_1, gsem_2, gsem_3,
                   ssem_0, ssem_1, ssem_2, ssem_3):
        cid = lax.axis_index("c")
        sid = lax.axis_index("s")
        wid = cid * NS + sid
        gl = jnp.where(cid == 0, gpt0, gpt1)
        base_chunk = jnp.where(cid == 0, sid * gpt0, NS * gpt0 + sid * gpt1)
        rqs = (rq_0, rq_1, rq_2, rq_3)
        gsems = (gsem_0, gsem_1, gsem_2, gsem_3)
        ssems = (ssem_0, ssem_1, ssem_2, ssem_3)

        zeros16 = jnp.zeros((L,), jnp.float32)

        @pl.loop(0, Q)
        def _(j):
            for c in range(D // L):
                rq_0[j, pl.ds(c * L, L)] = zeros16

        acc_rpt = NV // NS

        @pl.loop(0, acc_rpt // Q)
        def _(b):
            pltpu.sync_copy(rq_0, acc.at[pl.ds(sid * acc_rpt + b * Q, Q)])

        _rem = acc_rpt % Q
        if _rem:
            pltpu.sync_copy(
                rq_0.at[pl.ds(0, _rem)],
                acc.at[pl.ds(sid * acc_rpt + acc_rpt - _rem, _rem)])

        @pl.loop(0, NV // L)
        def _(j):
            s_v[pl.ds(j * L, L)] = zeros16

        pltpu.sync_copy(asrc_hbm.at[pl.ds(0, NV)], as_v)
        pltpu.sync_copy(adst_hbm.at[pl.ds(0, NV)], ad_v)
        pltpu.async_copy(idx_hbm.at[base_chunk], idx_0, isem_0)
        pltpu.async_copy(idx_hbm.at[base_chunk + 1], idx_1, isem_1)

        plsc.subcore_barrier()

        lanes = lax.iota(jnp.int32, L)

        def scalar_phase(g, ibuf):
            base = (base_chunk + g) * C
            for k in range(C // L):
                row, off = (k * L) // Q, (k * L) % Q
                sv = ibuf[row, pl.ds(off, L)]
                dv = ibuf[NQ + row, pl.ds(off, L)]
                av = plsc.load_gather(as_v, [sv])
                bv = plsc.load_gather(ad_v, [dv])
                e = av + bv
                e = jnp.where(e >= 0, e, e * jnp.float32(0.2))
                w = jnp.exp(e)
                valid = (base + k * L + lanes) < e_real
                w = jnp.where(valid, w, jnp.float32(0.0))
                w_buf[pl.ds(k * L, L)] = w
                plsc.addupdate_scatter(s_v, [dv], w)

        def multiply(i):
            @plsc.parallel_loop(0, Q, unroll=4)
            def _(j):
                wj = plsc.load_gather(w_buf, [jnp.broadcast_to(j + i * Q,
                                                               (L,))])
                for c in range(D // L):
                    sl = pl.ds(c * L, L)
                    rqs[i][j, sl] = rqs[i][j, sl] * wj

        def wait_scatter(i, ibuf):
            pltpu.make_async_copy(rqs[i], acc.at[ibuf.at[NQ + i]],
                                  ssems[i]).wait()

        def issue_gather(i, ibuf):
            pltpu.async_copy(h_hbm.at[ibuf.at[i]], rqs[i], gsems[i])

        def finish_quarter(i, ibuf):
            pltpu.make_async_copy(h_hbm.at[ibuf.at[i]], rqs[i],
                                  gsems[i]).wait()
            multiply(i)
            pltpu.async_copy(rqs[i], acc.at[ibuf.at[NQ + i]], ssems[i],
                             add=True)

        def process_chunk(g, ibuf, nbuf, nsem, pbuf, psem):
            scalar_phase(g, ibuf)

            @pl.when(g > 0)
            def _():
                wait_scatter(3, ibuf)
            issue_gather(3, ibuf)

            @pl.when(g + 2 < gl)
            def _():
                pltpu.async_copy(idx_hbm.at[base_chunk + g + 2], pbuf, psem)

            finish_quarter(0, ibuf)

            @pl.when(g + 1 < gl)
            def _():
                pltpu.make_async_copy(idx_hbm.at[base_chunk + g + 1], nbuf,
                                      nsem).wait()
                wait_scatter(0, nbuf)
                issue_gather(0, nbuf)

            finish_quarter(1, ibuf)

            @pl.when(g + 1 < gl)
            def _():
                wait_scatter(1, nbuf)
                issue_gather(1, nbuf)

            finish_quarter(2, ibuf)

            @pl.when(g + 1 < gl)
            def _():
                wait_scatter(2, nbuf)
                issue_gather(2, nbuf)

            finish_quarter(3, ibuf)

        pltpu.make_async_copy(idx_hbm.at[base_chunk], idx_0, isem_0).wait()
        issue_gather(0, idx_0)
        issue_gather(1, idx_0)
        issue_gather(2, idx_0)

        @pl.loop(0, gl, step=3)
        def _(g):
            process_chunk(g, idx_0, idx_1, isem_1, idx_2, isem_2)
            process_chunk(g + 1, idx_1, idx_2, isem_2, idx_0, isem_0)
            process_chunk(g + 2, idx_2, idx_0, isem_0, idx_1, isem_1)

        for i in range(NQ):
            wait_scatter(i, idx_2)

        plsc.subcore_barrier()

        pltpu.sync_copy(s_v, sp_hbm.at[wid].at[pl.ds(0, NV)])
        pltpu.sync_copy(acc.at[pl.ds(sid * acc_rpt, acc_rpt)],
                        accp_hbm.at[cid].at[pl.ds(sid * acc_rpt, acc_rpt)])

    return edge_phase


def kernel(x, edge_index, W1, a_src1, a_dst1, b1, W2, a_src2, a_dst2, b2):
    n = x.shape[0]
    e = edge_index.shape[1]
    e_real = e + n
    gpt = _cdiv(e_real, NW * C)
    gpt = _cdiv(gpt, 3) * 3
    tot_chunks = gpt * NW
    epad = tot_chunks * C
    gpt0 = (_cdiv(int(round(tot_chunks * 0.578 / NS)), 3) * 3)
    gpt1 = (tot_chunks - gpt0 * NS) // NS
    assert gpt1 % 3 == 0 and gpt1 > 0 and gpt0 * NS + gpt1 * NS == tot_chunks

    x_pad = jnp.pad(x, ((0, NPAD - n), (0, 0)))
    loop_idx = jnp.arange(n, dtype=edge_index.dtype)
    pad_idx = jnp.zeros((epad - e_real,), edge_index.dtype)
    src_t = jnp.concatenate([edge_index[0], loop_idx, pad_idx]).reshape(
        tot_chunks, NQ, Q)
    dst_t = jnp.concatenate([edge_index[1], loop_idx, pad_idx]).reshape(
        tot_chunks, NQ, Q)
    idx_t = jnp.concatenate([src_t, dst_t], axis=1)

    edge_phase = _make_edge_phase(e_real, gpt0, gpt1)

    h1, as1, ad1 = _project(x_pad, W1, a_src1.reshape(-1), a_dst1.reshape(-1))
    accp1, sp1 = edge_phase(h1, as1, ad1, idx_t)
    h2, as2, ad2 = _combine_project(accp1, sp1, b1, W2,
                                    a_src2.reshape(-1), a_dst2.reshape(-1))
    accp2, sp2 = edge_phase(h2, as2, ad2, idx_t)
    out = _final(accp2, sp2, b2)
    return out[:n]


# speedup vs baseline: 1.2363x; 1.0148x over previous
import dataclasses
import functools

import jax
import jax.numpy as jnp
from jax import lax
from jax.experimental import pallas as pl
from jax.experimental.pallas import tpu as pltpu
from jax.experimental.pallas import tpu_sc as plsc

N = 10000
D = 128
NPAD = 10240
NC, NS, L = 2, 16, 16
NW = NC * NS
C = 128
NQ = 4
Q = C // NQ
NV = 10112
ROWS_PER_TILE = NPAD // NS


def _cdiv(a, b):
    return (a + b - 1) // b


_GRID = 10
_RB = NPAD // _GRID


def _proj_body(x_ref, w_ref, av_s_ref, av_d_ref, h_ref, as_ref, ad_ref):
    h = jnp.dot(x_ref[...], w_ref[...], preferred_element_type=jnp.float32)
    h_ref[...] = h
    as_ref[...] = jnp.sum(h * av_s_ref[...][None, :], axis=1)
    ad_ref[...] = jnp.sum(h * av_d_ref[...][None, :], axis=1)


def _project(x, w, av_s, av_d):
    return pl.pallas_call(
        _proj_body,
        grid=(_GRID,),
        in_specs=[
            pl.BlockSpec((_RB, D), lambda i: (i, 0)),
            pl.BlockSpec((D, D), lambda i: (0, 0)),
            pl.BlockSpec((D,), lambda i: (0,)),
            pl.BlockSpec((D,), lambda i: (0,)),
        ],
        out_specs=[
            pl.BlockSpec((_RB, D), lambda i: (i, 0)),
            pl.BlockSpec((_RB,), lambda i: (i,)),
            pl.BlockSpec((_RB,), lambda i: (i,)),
        ],
        out_shape=[
            jax.ShapeDtypeStruct((NPAD, D), jnp.float32),
            jax.ShapeDtypeStruct((NPAD,), jnp.float32),
            jax.ShapeDtypeStruct((NPAD,), jnp.float32),
        ],
    )(x, w, av_s, av_d)


def _combine_body(accp_ref, sp_ref, b_ref, w_ref, av_s_ref, av_d_ref,
                  h_ref, as_ref, ad_ref):
    s = jnp.sum(sp_ref[...], axis=0)
    o = (accp_ref[0] + accp_ref[1]) / (s[:, None] + 1e-16) + b_ref[...][None, :]
    e = jnp.where(o > 0, o, jnp.exp(o) - 1.0)
    h = jnp.dot(e, w_ref[...], preferred_element_type=jnp.float32)
    h_ref[...] = h
    as_ref[...] = jnp.sum(h * av_s_ref[...][None, :], axis=1)
    ad_ref[...] = jnp.sum(h * av_d_ref[...][None, :], axis=1)


def _combine_project(accp, sp, b, w, av_s, av_d):
    return pl.pallas_call(
        _combine_body,
        grid=(_GRID,),
        in_specs=[
            pl.BlockSpec((NC, _RB, D), lambda i: (0, i, 0)),
            pl.BlockSpec((NW, _RB), lambda i: (0, i)),
            pl.BlockSpec((D,), lambda i: (0,)),
            pl.BlockSpec((D, D), lambda i: (0, 0)),
            pl.BlockSpec((D,), lambda i: (0,)),
            pl.BlockSpec((D,), lambda i: (0,)),
        ],
        out_specs=[
            pl.BlockSpec((_RB, D), lambda i: (i, 0)),
            pl.BlockSpec((_RB,), lambda i: (i,)),
            pl.BlockSpec((_RB,), lambda i: (i,)),
        ],
        out_shape=[
            jax.ShapeDtypeStruct((NPAD, D), jnp.float32),
            jax.ShapeDtypeStruct((NPAD,), jnp.float32),
            jax.ShapeDtypeStruct((NPAD,), jnp.float32),
        ],
    )(accp, sp, b, w, av_s, av_d)


def _final_body(accp_ref, sp_ref, b_ref, out_ref):
    s = jnp.sum(sp_ref[...], axis=0)
    out_ref[...] = ((accp_ref[0] + accp_ref[1]) / (s[:, None] + 1e-16)
                    + b_ref[...][None, :])


def _final(accp, sp, b):
    return pl.pallas_call(
        _final_body,
        grid=(_GRID,),
        in_specs=[
            pl.BlockSpec((NC, _RB, D), lambda i: (0, i, 0)),
            pl.BlockSpec((NW, _RB), lambda i: (0, i)),
            pl.BlockSpec((D,), lambda i: (0,)),
        ],
        out_specs=pl.BlockSpec((_RB, D), lambda i: (i, 0)),
        out_shape=jax.ShapeDtypeStruct((NPAD, D), jnp.float32),
    )(accp, sp, b)


def _make_edge_phase(e_real, gpt0, gpt1):
    mesh = plsc.VectorSubcoreMesh(core_axis_name="c", subcore_axis_name="s")
    cp = pltpu.CompilerParams()
    if "needs_layout_passes" in pltpu.CompilerParams.__dataclass_fields__:
        cp = dataclasses.replace(cp, needs_layout_passes=False)

    @functools.partial(
        pl.kernel,
        compiler_params=cp,
        out_type=[
            jax.ShapeDtypeStruct((NC, NPAD, D), jnp.float32),
            jax.ShapeDtypeStruct((NW, NPAD), jnp.float32),
        ],
        mesh=mesh,
        scratch_types=[
            pltpu.VMEM((NV,), jnp.float32),
            pltpu.VMEM((NV,), jnp.float32),
            pltpu.VMEM((NV,), jnp.float32),
            pltpu.VMEM((2 * NQ, Q), jnp.int32),
            pltpu.VMEM((2 * NQ, Q), jnp.int32),
            pltpu.VMEM((2 * NQ, Q), jnp.int32),
            pltpu.VMEM((C,), jnp.float32),
            pltpu.VMEM((Q, D), jnp.float32),
            pltpu.VMEM((Q, D), jnp.float32),
            pltpu.VMEM((Q, D), jnp.float32),
            pltpu.VMEM((Q, D), jnp.float32),
            pltpu.VMEM_SHARED((NV, D), jnp.float32),
            pltpu.SemaphoreType.DMA,
            pltpu.SemaphoreType.DMA,
            pltpu.SemaphoreType.DMA,
            pltpu.SemaphoreType.DMA,
            pltpu.SemaphoreType.DMA,
            pltpu.SemaphoreType.DMA,
            pltpu.SemaphoreType.DMA,
            pltpu.SemaphoreType.DMA,
            pltpu.SemaphoreType.DMA,
            pltpu.SemaphoreType.DMA,
            pltpu.SemaphoreType.DMA,
        ],
    )
    def edge_phase(h_hbm, asrc_hbm, adst_hbm, idx_hbm,
                   accp_hbm, sp_hbm,
                   as_v, ad_v, s_v, idx_0, idx_1, idx_2, w_buf,
                   rq_0, rq_1, rq_2, rq_3, acc,
                   isem_0, isem_1, isem_2,
                   gsem_0, gsem_1, gsem_2, gsem_3,
                   ssem_0, ssem_1, ssem_2, ssem_3):
        cid = lax.axis_index("c")
        sid = lax.axis_index("s")
        wid = cid * NS + sid
        gl = jnp.where(cid == 0, gpt0, gpt1)
        base_chunk = jnp.where(cid == 0, sid * gpt0, NS * gpt0 + sid * gpt1)
        rqs = (rq_0, rq_1, rq_2, rq_3)
        gsems = (gsem_0, gsem_1, gsem_2, gsem_3)
        ssems = (ssem_0, ssem_1, ssem_2, ssem_3)

        zeros16 = jnp.zeros((L,), jnp.float32)

        @pl.loop(0, Q)
        def _(j):
            for c in range(D // L):
                rq_0[j, pl.ds(c * L, L)] = zeros16

        acc_rpt = NV // NS

        @pl.loop(0, acc_rpt // Q)
        def _(b):
            pltpu.sync_copy(rq_0, acc.at[pl.ds(sid * acc_rpt + b * Q, Q)])

        _rem = acc_rpt % Q
        if _rem:
            pltpu.sync_copy(
                rq_0.at[pl.ds(0, _rem)],
                acc.at[pl.ds(sid * acc_rpt + acc_rpt - _rem, _rem)])

        @pl.loop(0, NV // L)
        def _(j):
            s_v[pl.ds(j * L, L)] = zeros16

        pltpu.sync_copy(asrc_hbm.at[pl.ds(0, NV)], as_v)
        pltpu.sync_copy(adst_hbm.at[pl.ds(0, NV)], ad_v)
        pltpu.async_copy(idx_hbm.at[base_chunk], idx_0, isem_0)
        pltpu.async_copy(idx_hbm.at[base_chunk + 1], idx_1, isem_1)

        plsc.subcore_barrier()

        lanes = lax.iota(jnp.int32, L)

        def scalar_phase(g, ibuf):
            base = (base_chunk + g) * C
            for k in range(C // L):
                row, off = (k * L) // Q, (k * L) % Q
                sv = ibuf[row, pl.ds(off, L)]
                dv = ibuf[NQ + row, pl.ds(off, L)]
                av = plsc.load_gather(as_v, [sv])
                bv = plsc.load_gather(ad_v, [dv])
                e = av + bv
                e = jnp.where(e >= 0, e, e * jnp.float32(0.2))
                w = jnp.exp(e)
                valid = (base + k * L + lanes) < e_real
                w = jnp.where(valid, w, jnp.float32(0.0))
                w_buf[pl.ds(k * L, L)] = w
                plsc.addupdate_scatter(s_v, [dv], w)

        def multiply(i):
            @plsc.parallel_loop(0, Q, unroll=4)
            def _(j):
                wj = plsc.load_gather(w_buf, [jnp.broadcast_to(j + i * Q,
                                                               (L,))])
                for c in range(D // L):
                    sl = pl.ds(c * L, L)
                    rqs[i][j, sl] = rqs[i][j, sl] * wj

        def wait_scatter(i, ibuf):
            pltpu.make_async_copy(rqs[i], acc.at[ibuf.at[NQ + i]],
                                  ssems[i]).wait()

        def issue_gather(i, ibuf):
            pltpu.async_copy(h_hbm.at[ibuf.at[i]], rqs[i], gsems[i])

        def finish_quarter(i, ibuf):
            pltpu.make_async_copy(h_hbm.at[ibuf.at[i]], rqs[i],
                                  gsems[i]).wait()
            multiply(i)
            pltpu.async_copy(rqs[i], acc.at[ibuf.at[NQ + i]], ssems[i],
                             add=True)

        def process_chunk(g, ibuf, nbuf, nsem, pbuf, psem):
            scalar_phase(g, ibuf)

            @pl.when(g > 0)
            def _():
                wait_scatter(3, ibuf)
            issue_gather(3, ibuf)

            @pl.when(g + 2 < gl)
            def _():
                pltpu.async_copy(idx_hbm.at[base_chunk + g + 2], pbuf, psem)

            finish_quarter(0, ibuf)

            @pl.when(g + 1 < gl)
            def _():
                pltpu.make_async_copy(idx_hbm.at[base_chunk + g + 1], nbuf,
                                      nsem).wait()
                wait_scatter(0, nbuf)
                issue_gather(0, nbuf)

            finish_quarter(1, ibuf)

            @pl.when(g + 1 < gl)
            def _():
                wait_scatter(1, nbuf)
                issue_gather(1, nbuf)

            finish_quarter(2, ibuf)

            @pl.when(g + 1 < gl)
            def _():
                wait_scatter(2, nbuf)
                issue_gather(2, nbuf)

            finish_quarter(3, ibuf)

        pltpu.make_async_copy(idx_hbm.at[base_chunk], idx_0, isem_0).wait()
        issue_gather(0, idx_0)
        issue_gather(1, idx_0)
        issue_gather(2, idx_0)

        @pl.loop(0, gl, step=3)
        def _(g):
            process_chunk(g, idx_0, idx_1, isem_1, idx_2, isem_2)
            process_chunk(g + 1, idx_1, idx_2, isem_2, idx_0, isem_0)
            process_chunk(g + 2, idx_2, idx_0, isem_0, idx_1, isem_1)

        for i in range(NQ):
            wait_scatter(i, idx_2)

        plsc.subcore_barrier()

        pltpu.sync_copy(s_v, sp_hbm.at[wid].at[pl.ds(0, NV)])
        pltpu.sync_copy(acc.at[pl.ds(sid * acc_rpt, acc_rpt)],
                        accp_hbm.at[cid].at[pl.ds(sid * acc_rpt, acc_rpt)])

    return edge_phase


def kernel(x, edge_index, W1, a_src1, a_dst1, b1, W2, a_src2, a_dst2, b2):
    n = x.shape[0]
    e = edge_index.shape[1]
    e_real = e + n
    gpt = _cdiv(e_real, NW * C)
    gpt = _cdiv(gpt, 3) * 3
    tot_chunks = gpt * NW
    epad = tot_chunks * C
    gpt0 = (_cdiv(int(round(tot_chunks * 0.605 / NS)), 3) * 3)
    gpt1 = (tot_chunks - gpt0 * NS) // NS
    assert gpt1 % 3 == 0 and gpt1 > 0 and gpt0 * NS + gpt1 * NS == tot_chunks

    x_pad = jnp.pad(x, ((0, NPAD - n), (0, 0)))
    loop_idx = jnp.arange(n, dtype=edge_index.dtype)
    pad_idx = jnp.zeros((epad - e_real,), edge_index.dtype)
    src_t = jnp.concatenate([edge_index[0], loop_idx, pad_idx]).reshape(
        tot_chunks, NQ, Q)
    dst_t = jnp.concatenate([edge_index[1], loop_idx, pad_idx]).reshape(
        tot_chunks, NQ, Q)
    idx_t = jnp.concatenate([src_t, dst_t], axis=1)

    edge_phase = _make_edge_phase(e_real, gpt0, gpt1)

    h1, as1, ad1 = _project(x_pad, W1, a_src1.reshape(-1), a_dst1.reshape(-1))
    accp1, sp1 = edge_phase(h1, as1, ad1, idx_t)
    h2, as2, ad2 = _combine_project(accp1, sp1, b1, W2,
                                    a_src2.reshape(-1), a_dst2.reshape(-1))
    accp2, sp2 = edge_phase(h2, as2, ad2, idx_t)
    out = _final(accp2, sp2, b2)
    return out[:n]
